# baseline (speedup 1.0000x reference)
import functools

import jax
import jax.numpy as jnp
from jax import lax
from jax.experimental import pallas as pl
from jax.experimental.pallas import tpu as pltpu
from jax.experimental.pallas import tpu_sc as plsc

G_SEL = 50000
EMBED_DIM = 64

NC, NS = 2, 16
NW = NC * NS
BK = 2048
K_PAD = 51200
BPW = K_PAD // NW
SUB = 100
NSUB = BPW // SUB
HALF = 2

_mesh = plsc.VectorSubcoreMesh(core_axis_name="c", subcore_axis_name="s")


@functools.partial(
    pl.kernel,
    mesh=_mesh,
    out_type=jax.ShapeDtypeStruct((K_PAD, 2 * EMBED_DIM), jnp.float32),
    scratch_types=[
        pltpu.VMEM((NSUB, SUB), jnp.int32),
        pltpu.VMEM((BPW // HALF, 2 * EMBED_DIM), jnp.float32),
        pltpu.SemaphoreType.DMA,
    ],
)
def _sc_gather(table_hbm, pair_hbm, out_hbm, idx_v, rows_v, sem):
    wid = lax.axis_index("s") * NC + lax.axis_index("c")
    base = wid * BPW
    pltpu.sync_copy(pair_hbm.at[pl.ds(wid * NSUB, NSUB)], idx_v)
    for h in range(HALF):
        copies = [
            pltpu.async_copy(
                table_hbm.at[idx_v.at[h * (NSUB // HALF) + j]],
                rows_v.at[pl.ds(j * SUB, SUB)],
                sem,
            )
            for j in range(NSUB // HALF)
        ]
        for c in copies:
            c.wait()
        dst = pl.ds(base + h * (BPW // HALF), BPW // HALF)
        pltpu.sync_copy(rows_v, out_hbm.at[dst])


def _mm_body(x_ref, g_ref, idx_ref, o_ref):
    k = pl.program_id(0)

    @pl.when(k == 0)
    def _():
        o_ref[...] = jnp.zeros_like(o_ref)

    xb = x_ref[...]
    col = k * BK + lax.broadcasted_iota(jnp.int32, (1, BK), 1)
    xb = jnp.where(col < G_SEL, xb, 0.0)
    par = (idx_ref[...] & 1).astype(jnp.float32)
    g = g_ref[...]
    acc = jnp.dot(xb * (1.0 - par), g[:, :EMBED_DIM],
                  preferred_element_type=jnp.float32)
    acc += jnp.dot(xb * par, g[:, EMBED_DIM:],
                   preferred_element_type=jnp.float32)
    o_ref[...] += acc


def _tc_matmul(x, g, idx_row):
    grid = K_PAD // BK
    return pl.pallas_call(
        _mm_body,
        grid=(grid,),
        in_specs=[
            pl.BlockSpec((x.shape[0], BK), lambda k: (0, k)),
            pl.BlockSpec((BK, 2 * EMBED_DIM), lambda k: (k, 0)),
            pl.BlockSpec((1, BK), lambda k: (0, k)),
        ],
        out_specs=pl.BlockSpec((x.shape[0], EMBED_DIM), lambda k: (0, 0)),
        out_shape=jax.ShapeDtypeStruct((x.shape[0], EMBED_DIM), jnp.float32),
        compiler_params=pltpu.CompilerParams(
            dimension_semantics=("arbitrary",),
        ),
    )(x, g, idx_row)


def kernel(x, gene_idx, gene_embeddings):
    table2 = gene_embeddings.reshape(gene_embeddings.shape[0] // 2,
                                     2 * EMBED_DIM)
    idx_pad = jnp.concatenate(
        [gene_idx, jnp.zeros((K_PAD - G_SEL,), jnp.int32)]
    )
    pair2d = (idx_pad >> 1).reshape(K_PAD // SUB, SUB)
    g = _sc_gather(table2, pair2d)
    return _tc_matmul(x, g, idx_pad.reshape(1, K_PAD))


# speedup vs baseline: 1.0041x; 1.0041x over previous
import functools

import jax
import jax.numpy as jnp
from jax import lax
from jax.experimental import pallas as pl
from jax.experimental.pallas import tpu as pltpu
from jax.experimental.pallas import tpu_sc as plsc

G_SEL = 50000
EMBED_DIM = 64
BATCH = 256

NC, NS = 2, 16
NW = NC * NS
BK = 2048
K_PAD = 51200
BPW = K_PAD // NW
SUB = 100
NSUB = BPW // SUB
HALF = 2

_mesh = plsc.VectorSubcoreMesh(core_axis_name="c", subcore_axis_name="s")


@functools.partial(
    pl.kernel,
    mesh=_mesh,
    out_type=jax.ShapeDtypeStruct((K_PAD, 2 * EMBED_DIM), jnp.float32),
    scratch_types=[
        pltpu.VMEM((NSUB, SUB), jnp.int32),
        pltpu.VMEM((BPW // HALF, 2 * EMBED_DIM), jnp.float32),
        pltpu.SemaphoreType.DMA,
    ],
)
def _sc_gather(table_hbm, pair_hbm, out_hbm, idx_v, rows_v, sem):
    wid = lax.axis_index("s") * NC + lax.axis_index("c")
    base = wid * BPW
    pltpu.sync_copy(pair_hbm.at[pl.ds(wid * NSUB, NSUB)], idx_v)
    for h in range(HALF):
        copies = [
            pltpu.async_copy(
                table_hbm.at[idx_v.at[h * (NSUB // HALF) + j]],
                rows_v.at[pl.ds(j * SUB, SUB)],
                sem,
            )
            for j in range(NSUB // HALF)
        ]
        for c in copies:
            c.wait()
        dst = pl.ds(base + h * (BPW // HALF), BPW // HALF)
        pltpu.sync_copy(rows_v, out_hbm.at[dst])


def _mm_body(g_ref, xt_ref, idx_ref, o_ref):
    k = pl.program_id(0)

    @pl.when(k == 0)
    def _():
        o_ref[...] = jnp.zeros_like(o_ref)

    row = k * BK + lax.broadcasted_iota(jnp.int32, (BK, 1), 0)
    xb = jnp.where(row < G_SEL, xt_ref[...], 0.0)
    par = (idx_ref[...] & 1).astype(jnp.float32)
    gb = g_ref[...]
    acc = lax.dot_general(
        gb[:, :EMBED_DIM], xb * (1.0 - par),
        (((0,), (0,)), ((), ())),
        preferred_element_type=jnp.float32,
    )
    acc += lax.dot_general(
        gb[:, EMBED_DIM:], xb * par,
        (((0,), (0,)), ((), ())),
        preferred_element_type=jnp.float32,
    )
    o_ref[...] += acc


def _tc_matmul(g, xt, idx_col):
    grid = K_PAD // BK
    return pl.pallas_call(
        _mm_body,
        grid=(grid,),
        in_specs=[
            pl.BlockSpec((BK, 2 * EMBED_DIM), lambda k: (k, 0)),
            pl.BlockSpec((BK, BATCH), lambda k: (k, 0)),
            pl.BlockSpec((BK, 1), lambda k: (k, 0)),
        ],
        out_specs=pl.BlockSpec((EMBED_DIM, BATCH), lambda k: (0, 0)),
        out_shape=jax.ShapeDtypeStruct((EMBED_DIM, BATCH), jnp.float32),
        compiler_params=pltpu.CompilerParams(
            dimension_semantics=("arbitrary",),
        ),
    )(g, xt, idx_col)


def kernel(x, gene_idx, gene_embeddings):
    tbl = jnp.reshape(
        gene_embeddings, (gene_embeddings.shape[0] // 2, 2 * EMBED_DIM)
    )
    idx_pad = jnp.concatenate(
        [gene_idx, jnp.zeros((K_PAD - G_SEL,), jnp.int32)]
    )
    pair2d = (idx_pad >> 1).reshape(K_PAD // SUB, SUB)
    g = _sc_gather(tbl, pair2d)
    out_t = _tc_matmul(g, x.T, idx_pad.reshape(K_PAD, 1))
    return out_t.T


# speedup vs baseline: 1.1245x; 1.1199x over previous
import functools

import jax
import jax.numpy as jnp
from jax import lax
from jax.experimental import pallas as pl
from jax.experimental.pallas import tpu as pltpu
from jax.experimental.pallas import tpu_sc as plsc

G_SEL = 50000
EMBED_DIM = 64
BATCH = 256
ROW_PAD = 128

NC, NS = 2, 16
NW = NC * NS
BK = 2048
K_PAD = 51200
BPW = K_PAD // NW
SUB = 100
NSUB = BPW // SUB
HALF = 2

_mesh = plsc.VectorSubcoreMesh(core_axis_name="c", subcore_axis_name="s")


@functools.partial(
    pl.kernel,
    mesh=_mesh,
    out_type=jax.ShapeDtypeStruct((K_PAD, ROW_PAD), jnp.float32),
    scratch_types=[
        pltpu.VMEM((NSUB, SUB), jnp.int32),
        pltpu.VMEM((BPW // HALF, ROW_PAD), jnp.float32),
        pltpu.SemaphoreType.DMA,
    ],
)
def _sc_gather(table_hbm, idx2_hbm, out_hbm, idx_v, rows_v, sem):
    wid = lax.axis_index("s") * NC + lax.axis_index("c")
    base = wid * BPW
    pltpu.sync_copy(idx2_hbm.at[pl.ds(wid * NSUB, NSUB)], idx_v)
    for h in range(HALF):
        copies = [
            pltpu.async_copy(
                table_hbm.at[idx_v.at[h * (NSUB // HALF) + j]],
                rows_v.at[pl.ds(j * SUB, SUB)],
                sem,
            )
            for j in range(NSUB // HALF)
        ]
        for c in copies:
            c.wait()
        dst = pl.ds(base + h * (BPW // HALF), BPW // HALF)
        pltpu.sync_copy(rows_v, out_hbm.at[dst])


def _mm_body(g_ref, xt_ref, o_ref):
    k = pl.program_id(0)

    @pl.when(k == 0)
    def _():
        o_ref[...] = jnp.zeros_like(o_ref)

    row = k * BK + lax.broadcasted_iota(jnp.int32, (BK, 1), 0)
    xb = jnp.where(row < G_SEL, xt_ref[...], 0.0)
    gb = g_ref[...][:, :EMBED_DIM]
    o_ref[...] += lax.dot_general(
        gb, xb, (((0,), (0,)), ((), ())),
        preferred_element_type=jnp.float32,
    )


def _tc_matmul(g, xt):
    grid = K_PAD // BK
    return pl.pallas_call(
        _mm_body,
        grid=(grid,),
        in_specs=[
            pl.BlockSpec((BK, ROW_PAD), lambda k: (k, 0)),
            pl.BlockSpec((BK, BATCH), lambda k: (k, 0)),
        ],
        out_specs=pl.BlockSpec((EMBED_DIM, BATCH), lambda k: (0, 0)),
        out_shape=jax.ShapeDtypeStruct((EMBED_DIM, BATCH), jnp.float32),
        compiler_params=pltpu.CompilerParams(
            dimension_semantics=("arbitrary",),
        ),
    )(g, xt)


def kernel(x, gene_idx, gene_embeddings):
    tbl = jnp.pad(gene_embeddings, ((0, 0), (0, ROW_PAD - EMBED_DIM)))
    idx_pad = jnp.concatenate(
        [gene_idx, jnp.zeros((K_PAD - G_SEL,), jnp.int32)]
    )
    idx2d = idx_pad.reshape(K_PAD // SUB, SUB)
    g = _sc_gather(tbl, idx2d)
    out_t = _tc_matmul(g, x.T)
    return out_t.T


# speedup vs baseline: 1.6330x; 1.4522x over previous
import functools

import jax
import jax.numpy as jnp
from jax import lax
from jax.experimental import pallas as pl
from jax.experimental.pallas import tpu as pltpu
from jax.experimental.pallas import tpu_sc as plsc

G_SEL = 50000
EMBED_DIM = 64
BATCH = 256
ROW_PAD = 128

NC, NS = 2, 16
NW = NC * NS
BK = 2048
K_PAD = 51200
BPW = K_PAD // NW
SUB = 100
NSUB = BPW // SUB
HALF = 2

_mesh = plsc.VectorSubcoreMesh(core_axis_name="c", subcore_axis_name="s")


@functools.partial(
    pl.kernel,
    mesh=_mesh,
    out_type=jax.ShapeDtypeStruct((K_PAD, ROW_PAD), jnp.float32),
    scratch_types=[
        pltpu.VMEM((NSUB, SUB), jnp.int32),
        pltpu.VMEM((BPW // HALF, ROW_PAD), jnp.float32),
        pltpu.SemaphoreType.DMA,
    ],
)
def _sc_gather(table_hbm, idx2_hbm, out_hbm, idx_v, rows_v, sem):
    wid = lax.axis_index("s") * NC + lax.axis_index("c")
    base = wid * BPW
    pltpu.sync_copy(idx2_hbm.at[pl.ds(wid * NSUB, NSUB)], idx_v)
    for h in range(HALF):
        copies = [
            pltpu.async_copy(
                table_hbm.at[idx_v.at[h * (NSUB // HALF) + j]],
                rows_v.at[pl.ds(j * SUB, SUB)],
                sem,
            )
            for j in range(NSUB // HALF)
        ]
        for c in copies:
            c.wait()
        dst = pl.ds(base + h * (BPW // HALF), BPW // HALF)
        pltpu.sync_copy(rows_v, out_hbm.at[dst])


BKV = 4096
NVB = 244
V_TAIL = NVB * BKV


def _pad_body(in_ref, o_ref):
    t = jnp.transpose(in_ref[...])
    o_ref[...] = jnp.concatenate([t, jnp.zeros_like(t)], axis=1)


def _pad_relayout(table_t, vocab):
    return pl.pallas_call(
        _pad_body,
        grid=(NVB,),
        in_specs=[pl.BlockSpec((EMBED_DIM, BKV), lambda k: (0, k))],
        out_specs=pl.BlockSpec((BKV, ROW_PAD), lambda k: (k, 0)),
        out_shape=jax.ShapeDtypeStruct((vocab, ROW_PAD), jnp.float32),
        compiler_params=pltpu.CompilerParams(
            dimension_semantics=("parallel",),
        ),
    )(table_t)


def _mm_body(g_ref, xt_ref, o_ref):
    k = pl.program_id(0)

    @pl.when(k == 0)
    def _():
        o_ref[...] = jnp.zeros_like(o_ref)

    row = k * BK + lax.broadcasted_iota(jnp.int32, (BK, 1), 0)
    xb = jnp.where(row < G_SEL, xt_ref[...], 0.0)
    gb = g_ref[...][:, :EMBED_DIM]
    o_ref[...] += lax.dot_general(
        gb, xb, (((0,), (0,)), ((), ())),
        preferred_element_type=jnp.float32,
    )


def _tc_matmul(g, xt):
    grid = K_PAD // BK
    return pl.pallas_call(
        _mm_body,
        grid=(grid,),
        in_specs=[
            pl.BlockSpec((BK, ROW_PAD), lambda k: (k, 0)),
            pl.BlockSpec((BK, BATCH), lambda k: (k, 0)),
        ],
        out_specs=pl.BlockSpec((EMBED_DIM, BATCH), lambda k: (0, 0)),
        out_shape=jax.ShapeDtypeStruct((EMBED_DIM, BATCH), jnp.float32),
        compiler_params=pltpu.CompilerParams(
            dimension_semantics=("arbitrary",),
        ),
    )(g, xt)


def kernel(x, gene_idx, gene_embeddings):
    vocab = gene_embeddings.shape[0]
    tbl = _pad_relayout(gene_embeddings.T, vocab)
    tail = jnp.pad(
        gene_embeddings[V_TAIL:, :], ((0, 0), (0, ROW_PAD - EMBED_DIM))
    )
    tbl = lax.dynamic_update_slice(tbl, tail, (V_TAIL, 0))
    idx_pad = jnp.concatenate(
        [gene_idx, jnp.zeros((K_PAD - G_SEL,), jnp.int32)]
    )
    idx2d = idx_pad.reshape(K_PAD // SUB, SUB)
    g = _sc_gather(tbl, idx2d)
    out_t = _tc_matmul(g, x.T)
    return out_t.T
